# baseline (speedup 1.0000x reference)
import functools

import jax
import jax.numpy as jnp
from jax import lax
from jax.experimental import pallas as pl
from jax.experimental.pallas import tpu as pltpu
from jax.experimental.pallas import tpu_sc as plsc

N = 100000
HID = 128
VOCAB = 21
CHUNK = 128
NW = 32
NBUF = 2
N_CHUNKS = 782
LAST_BASE = N - CHUNK
N_SLOTS = 26

_mesh = plsc.VectorSubcoreMesh(core_axis_name="c", subcore_axis_name="s")


@functools.partial(
    pl.kernel,
    mesh=_mesh,
    out_type=jax.ShapeDtypeStruct((N, HID), jnp.float32),
    scratch_types=[
        pltpu.VMEM_SHARED((VOCAB, HID), jnp.float32),
        pltpu.VMEM((NBUF, CHUNK), jnp.int32),
        pltpu.VMEM((NBUF, CHUNK, HID), jnp.float32),
    ]
    + [pltpu.SemaphoreType.DMA] * (2 * NBUF),
)
def _embed(table_hbm, idx_hbm, out_hbm, tab_sh, idx_v, rows_v, *sems):
    sg = sems[:NBUF]
    sw = sems[NBUF:]
    sid = lax.axis_index("s")
    wid = sid * 2 + lax.axis_index("c")

    @pl.when(sid == 0)
    def _():
        pltpu.sync_copy(table_hbm, tab_sh)

    plsc.subcore_barrier()

    def valid(i):
        return wid + NW * i < N_CHUNKS

    def cbase(i):
        c = wid + NW * i
        return pl.multiple_of(jnp.minimum(c * CHUNK, LAST_BASE), 8)

    def stage_and_fire(i, b):
        base = cbase(i)
        pltpu.sync_copy(idx_hbm.at[pl.ds(base, CHUNK)], idx_v.at[b])
        pltpu.async_copy(tab_sh.at[idx_v.at[b]], rows_v.at[b], sg[b])

    def wait_gather(b):
        pltpu.make_async_copy(tab_sh.at[idx_v.at[b]], rows_v.at[b], sg[b]).wait()

    def fire_write(i, b):
        pltpu.async_copy(rows_v.at[b], out_hbm.at[pl.ds(cbase(i), CHUNK)], sw[b])

    def wait_write(b):
        pltpu.make_async_copy(rows_v.at[b], out_hbm.at[pl.ds(0, CHUNK)], sw[b]).wait()

    for b in range(NBUF):
        stage_and_fire(b, b)

    def pair(p, carry):
        for b in range(NBUF):
            i = NBUF * p + b

            @pl.when(valid(i))
            def _():
                wait_gather(b)
                fire_write(i, b)

        for b in range(NBUF):
            i = NBUF * p + b

            @pl.when(valid(i))
            def _():
                wait_write(b)

            @pl.when(valid(i + NBUF))
            def _():
                stage_and_fire(i + NBUF, b)

        return carry

    lax.fori_loop(0, N_SLOTS // NBUF, pair, 0)


def kernel(x, table):
    return _embed(table, x.astype(jnp.int32))


# speedup vs baseline: 1.0555x; 1.0555x over previous
import functools

import jax
import jax.numpy as jnp
from jax import lax
from jax.experimental import pallas as pl
from jax.experimental.pallas import tpu as pltpu
from jax.experimental.pallas import tpu_sc as plsc

N = 100000
HID = 128
VOCAB = 21
CHUNK = 128
NW = 32
NBUF = 4
N_CHUNKS = 782
LAST_BASE = N - CHUNK
MAIN_STEPS = 24

_mesh = plsc.VectorSubcoreMesh(core_axis_name="c", subcore_axis_name="s")


@functools.partial(
    pl.kernel,
    mesh=_mesh,
    out_type=jax.ShapeDtypeStruct((N, HID), jnp.float32),
    scratch_types=[
        pltpu.VMEM_SHARED((VOCAB, HID), jnp.float32),
        pltpu.VMEM((NBUF, CHUNK), jnp.int32),
        pltpu.VMEM((NBUF, CHUNK, HID), jnp.float32),
    ]
    + [pltpu.SemaphoreType.DMA] * (2 * NBUF),
)
def _embed(table_hbm, idx_hbm, out_hbm, tab_sh, idx_v, rows_v, *sems):
    sg = sems[:NBUF]
    sw = sems[NBUF:]
    sid = lax.axis_index("s")
    wid = sid * 2 + lax.axis_index("c")

    @pl.when(sid == 0)
    def _():
        pltpu.sync_copy(table_hbm, tab_sh)

    plsc.subcore_barrier()

    def cbase(i):
        c = wid + NW * i
        return pl.multiple_of(jnp.minimum(c * CHUNK, LAST_BASE), 8)

    def stage_and_fire(i, b):
        base = cbase(i)
        pltpu.sync_copy(idx_hbm.at[pl.ds(base, CHUNK)], idx_v.at[b])
        pltpu.async_copy(tab_sh.at[idx_v.at[b]], rows_v.at[b], sg[b])

    def wait_gather(b):
        pltpu.make_async_copy(tab_sh.at[idx_v.at[b]], rows_v.at[b], sg[b]).wait()

    def fire_write(i, b):
        pltpu.async_copy(rows_v.at[b], out_hbm.at[pl.ds(cbase(i), CHUNK)], sw[b])

    def wait_write(b):
        pltpu.make_async_copy(rows_v.at[b], out_hbm.at[pl.ds(0, CHUNK)], sw[b]).wait()

    for b in range(NBUF):
        stage_and_fire(b, b)

    def quad(p, carry):
        for b in range(NBUF):
            wait_gather(b)
            fire_write(NBUF * p + b, b)
        for b in range(NBUF):
            nxt = NBUF * (p + 1) + b

            @pl.when(nxt < MAIN_STEPS)
            def _():
                wait_write(b)
                stage_and_fire(nxt, b)

        return carry

    lax.fori_loop(0, MAIN_STEPS // NBUF, quad, 0)

    has_extra = wid < N_CHUNKS - NW * MAIN_STEPS

    @pl.when(has_extra)
    def _():
        wait_write(0)
        stage_and_fire(MAIN_STEPS, 0)
        wait_gather(0)
        fire_write(MAIN_STEPS, 0)
        wait_write(0)

    @pl.when(jnp.logical_not(has_extra))
    def _():
        wait_write(0)

    for b in range(1, NBUF):
        wait_write(b)


def kernel(x, table):
    return _embed(table, x.astype(jnp.int32))


# speedup vs baseline: 1.0557x; 1.0002x over previous
import functools

import jax
import jax.numpy as jnp
from jax import lax
from jax.experimental import pallas as pl
from jax.experimental.pallas import tpu as pltpu
from jax.experimental.pallas import tpu_sc as plsc

N = 100000
HID = 128
VOCAB = 21
CHUNK = 128
NW = 32
NBUF = 4
N_CHUNKS = 782
LAST_BASE = N - CHUNK
MAIN_STEPS = 24

_mesh = plsc.VectorSubcoreMesh(core_axis_name="c", subcore_axis_name="s")


@functools.partial(
    pl.kernel,
    mesh=_mesh,
    out_type=jax.ShapeDtypeStruct((N, HID), jnp.float32),
    scratch_types=[
        pltpu.VMEM_SHARED((VOCAB, HID), jnp.float32),
        pltpu.VMEM((NBUF, CHUNK), jnp.int32),
        pltpu.VMEM((NBUF, CHUNK, HID), jnp.float32),
    ]
    + [pltpu.SemaphoreType.DMA] * (2 * NBUF),
)
def _embed(table_hbm, idx_hbm, out_hbm, tab_sh, idx_v, rows_v, *sems):
    sg = sems[:NBUF]
    sw = sems[NBUF:]
    sid = lax.axis_index("s")
    wid = sid * 2 + lax.axis_index("c")

    @pl.when(sid == 0)
    def _():
        pltpu.sync_copy(table_hbm, tab_sh)

    plsc.subcore_barrier()

    def valid(i):
        return wid + NW * i < N_CHUNKS

    def cbase(i):
        c = wid + NW * i
        return pl.multiple_of(jnp.minimum(c * CHUNK, LAST_BASE), 8)

    def stage_and_fire(i, b):
        base = cbase(i)
        pltpu.sync_copy(idx_hbm.at[pl.ds(base, CHUNK)], idx_v.at[b])
        pltpu.async_copy(tab_sh.at[idx_v.at[b]], rows_v.at[b], sg[b])

    def wait_gather(b):
        pltpu.make_async_copy(tab_sh.at[idx_v.at[b]], rows_v.at[b], sg[b]).wait()

    def fire_write(i, b):
        pltpu.async_copy(rows_v.at[b], out_hbm.at[pl.ds(cbase(i), CHUNK)], sw[b])

    def wait_write(b):
        pltpu.make_async_copy(rows_v.at[b], out_hbm.at[pl.ds(0, CHUNK)], sw[b]).wait()

    for b in range(NBUF):
        stage_and_fire(b, b)

    def quad(p, carry):
        for b in range(NBUF):
            i = NBUF * p + b

            @pl.when(valid(i))
            def _():
                wait_gather(b)
                fire_write(i, b)

        for b in range(NBUF):
            i = NBUF * p + b

            @pl.when(valid(i))
            def _():
                wait_write(b)

            @pl.when(valid(i + NBUF))
            def _():
                stage_and_fire(i + NBUF, b)

        return carry

    lax.fori_loop(0, (MAIN_STEPS + NBUF) // NBUF, quad, 0)


def kernel(x, table):
    return _embed(table, x.astype(jnp.int32))
